# speedup vs baseline: 3.2914x; 3.2914x over previous
import numpy as np
import jax
import jax.numpy as jnp
from jax.experimental import pallas as pl
from jax.experimental.pallas import tpu as pltpu

_B, _T, _N, _H = 4, 64, 111, 111
_L = 4
_HEADS = 3
_HD = _N // _HEADS
_G = _B * _T
_GB = 8
_NBLK = _G // _GB
_ROWS = _G * _N
_M = _N * _N
_K = 8624
_EPS_BN = 1e-5

_f32 = jnp.float32


def _dotT(x, w):
    return jax.lax.dot_general(x, w, (((1,), (1,)), ((), ())),
                               preferred_element_type=_f32)


def _dot(x, w):
    return jax.lax.dot_general(x, w, (((1,), (0,)), ((), ())),
                               preferred_element_type=_f32)


def _stage_a_kernel(a_ref, v_ref, w_ref, b_ref, adj_ref, h0_ref):
    a = a_ref[...]
    i32 = jax.lax.bitcast_convert_type(a, jnp.int32)
    sign = jnp.int32(-2147483648)
    wb = jnp.where(i32 < 0, ~i32, i32 ^ sign)
    active = jnp.ones(a.shape, jnp.int32)
    k = jnp.full((_GB, 1, 1), _K, jnp.int32)
    p = jnp.zeros((_GB, 1, 1), jnp.int32)
    for bit in range(31, -1, -1):
        bitv = jax.lax.shift_right_logical(wb, bit) & 1
        zeros_here = active * (1 - bitv)
        c0 = jnp.sum(jnp.sum(zeros_here, axis=2, keepdims=True),
                     axis=1, keepdims=True)
        go1 = k >= c0
        bitval = jnp.int32(np.int32(np.uint32(1 << bit)))
        p = jnp.where(go1, p | bitval, p)
        k = jnp.where(go1, k - c0, k)
        want = jnp.where(go1, 1, 0)
        active = active * (bitv == want).astype(jnp.int32)
    key = p ^ sign
    thr_i = jnp.where(key < 0, key ^ jnp.int32(0x7FFFFFFF), key)
    thr = jax.lax.bitcast_convert_type(thr_i, _f32)
    adj_ref[...] = (a > thr).astype(_f32)
    w = w_ref[...]
    b = b_ref[...]
    for g in range(_GB):
        h0_ref[g * _N:(g + 1) * _N, :] = _dotT(v_ref[g], w) + b


def _gin1_kernel(adj_ref, h_ref, eps_ref, w1_ref, b1_ref, z_ref, s_ref, q_ref):
    h = h_ref[...]
    parts = []
    for g in range(_GB):
        hg = h[g * _N:(g + 1) * _N, :]
        parts.append(_dot(adj_ref[g], hg))
    agg = jnp.concatenate(parts, axis=0) + eps_ref[...] * h
    z = _dotT(agg, w1_ref[...]) + b1_ref[...]
    z_ref[...] = z
    s = jnp.sum(z, axis=0, keepdims=True)
    q = jnp.sum(z * z, axis=0, keepdims=True)

    @pl.when(pl.program_id(0) == 0)
    def _init():
        s_ref[...] = s
        q_ref[...] = q

    @pl.when(pl.program_id(0) != 0)
    def _acc():
        s_ref[...] += s
        q_ref[...] += q


def _gin2_kernel(z1_ref, s_ref, q_ref, g1_ref, be1_ref, w2_ref, b2_ref,
                 z2_ref, s2_ref, q2_ref):
    m = s_ref[...] * (1.0 / _ROWS)
    v = q_ref[...] * (1.0 / _ROWS) - m * m
    y = (z1_ref[...] - m) / jnp.sqrt(v + _EPS_BN) * g1_ref[...] + be1_ref[...]
    y = jnp.maximum(y, 0.0)
    z2 = _dotT(y, w2_ref[...]) + b2_ref[...]
    z2_ref[...] = z2
    s = jnp.sum(z2, axis=0, keepdims=True)
    q = jnp.sum(z2 * z2, axis=0, keepdims=True)

    @pl.when(pl.program_id(0) == 0)
    def _init():
        s2_ref[...] = s
        q2_ref[...] = q

    @pl.when(pl.program_id(0) != 0)
    def _acc():
        s2_ref[...] += s
        q2_ref[...] += q


def _gin3_kernel(z2_ref, s_ref, q_ref, g2_ref, be2_ref,
                 h_ref, xr_ref, hm_ref, ro_ref):
    m = s_ref[...] * (1.0 / _ROWS)
    v = q_ref[...] * (1.0 / _ROWS) - m * m
    h = (z2_ref[...] - m) / jnp.sqrt(v + _EPS_BN) * g2_ref[...] + be2_ref[...]
    h = jnp.maximum(h, 0.0)
    h_ref[...] = h

    row = jax.lax.broadcasted_iota(jnp.int32, (_N, _N), 0)
    col = jax.lax.broadcasted_iota(jnp.int32, (_N, _N), 1)
    upper = col >= row
    eye = (col == row).astype(_f32)
    ones_row = jnp.ones((1, _N), _f32)

    xrs = []
    hms = []
    ro = jnp.zeros((1, 1), _f32)
    for g in range(_GB):
        hg = h[g * _N:(g + 1) * _N, :]
        xrs.append(_dot(ones_row, hg) * (1.0 / _N))
        hms.append(_dotT(ones_row, hg) * (1.0 / _N))
        mi = _dotT(hg, hg)
        mx = jnp.max(mi, axis=1, keepdims=True)
        mi_n = mi / mx
        diff = jnp.where(upper, mi_n - eye, 0.0)
        ssq = jnp.sum(jnp.sum(diff * diff, axis=1, keepdims=True),
                      axis=0, keepdims=True)
        ro = ro + jnp.sqrt(ssq)
    xr_ref[...] = jnp.concatenate(xrs, axis=0)
    hm_ref[...] = jnp.concatenate(hms, axis=0)

    @pl.when(pl.program_id(0) == 0)
    def _init():
        ro_ref[...] = ro

    @pl.when(pl.program_id(0) != 0)
    def _acc():
        ro_ref[...] += ro


def _sero_kernel(xr_ref, hm_ref, ew_ref, eb_ref, g_ref, be_ref,
                 aw_ref, ab_ref, hro_ref):
    x = _dotT(xr_ref[...], ew_ref[...]) + eb_ref[...]
    m = jnp.mean(x, axis=0, keepdims=True)
    v = jnp.mean((x - m) * (x - m), axis=0, keepdims=True)
    x = (x - m) / jnp.sqrt(v + _EPS_BN) * g_ref[...] + be_ref[...]
    x = 0.5 * x * (1.0 + jax.lax.erf(x * np.float32(1.0 / np.sqrt(2.0))))
    gatt = jax.nn.sigmoid(_dotT(x, aw_ref[...]) + ab_ref[...])
    hro_ref[...] = gatt * hm_ref[...]


def _tr_kernel(x_ref,
               wq0_ref, wq1_ref, wq2_ref, wk0_ref, wk1_ref, wk2_ref,
               wv0_ref, wv1_ref, wv2_ref,
               bq0_ref, bq1_ref, bq2_ref, bk0_ref, bk1_ref, bk2_ref,
               bv0_ref, bv1_ref, bv2_ref,
               ow0_ref, ow1_ref, ow2_ref, ob_ref,
               ln1g_ref, ln1b_ref, m1w_ref, m1b_ref, m2w_ref, m2b_ref,
               ln2g_ref, ln2b_ref, cw_ref, cb_ref,
               lat_ref, log_ref):
    wq = (wq0_ref[...], wq1_ref[...], wq2_ref[...])
    wk = (wk0_ref[...], wk1_ref[...], wk2_ref[...])
    wv = (wv0_ref[...], wv1_ref[...], wv2_ref[...])
    bq = (bq0_ref[...], bq1_ref[...], bq2_ref[...])
    bk = (bk0_ref[...], bk1_ref[...], bk2_ref[...])
    bv = (bv0_ref[...], bv1_ref[...], bv2_ref[...])
    ow = (ow0_ref[...], ow1_ref[...], ow2_ref[...])
    scale = np.float32(1.0 / np.sqrt(_HD))

    def _ln(x, g, b):
        m = jnp.mean(x, axis=1, keepdims=True)
        v = jnp.mean((x - m) * (x - m), axis=1, keepdims=True)
        return (x - m) / jnp.sqrt(v + _EPS_BN) * g + b

    for b in range(_B):
        x = x_ref[b * _T:(b + 1) * _T, :]
        att = jnp.zeros((_T, _N), _f32)
        for hd in range(_HEADS):
            q = (_dotT(x, wq[hd]) + bq[hd]) * scale
            kk = _dotT(x, wk[hd]) + bk[hd]
            vv = _dotT(x, wv[hd]) + bv[hd]
            s = _dotT(q, kk)
            mx = jnp.max(s, axis=1, keepdims=True)
            e = jnp.exp(s - mx)
            pattn = e / jnp.sum(e, axis=1, keepdims=True)
            o = _dot(pattn, vv)
            att = att + _dotT(o, ow[hd])
        att = att + ob_ref[...]
        x1 = _ln(att, ln1g_ref[...], ln1b_ref[...])
        x2 = jnp.maximum(_dotT(x1, m1w_ref[...]) + m1b_ref[...], 0.0)
        x2 = _dotT(x2, m2w_ref[...]) + m2b_ref[...]
        xo = _ln(x1 + x2, ln2g_ref[...], ln2b_ref[...])
        lat = jnp.sum(xo, axis=0, keepdims=True)
        lat_ref[b:b + 1, :] = lat
        log_ref[b:b + 1, :] = _dotT(lat, cw_ref[...]) + cb_ref[...]


def _full_spec(shape):
    nd = len(shape)
    return pl.BlockSpec(shape, lambda *_, _nd=nd: (0,) * _nd)


_SEQ = pltpu.CompilerParams(dimension_semantics=("arbitrary",))


def kernel(v, a, init_w, init_b, gin_eps, gin_w1, gin_b1, gin_g1, gin_be1,
           gin_w2, gin_b2, gin_g2, gin_be2, sero_ew, sero_eb, sero_g, sero_be,
           sero_aw, sero_ab, tr_inw, tr_inb, tr_ow, tr_ob, tr_ln1g, tr_ln1b,
           tr_ln2g, tr_ln2b, tr_m1w, tr_m1b, tr_m2w, tr_m2b, cls_w, cls_b):
    a3 = a.reshape(_G, _N, _N)
    v3 = v.reshape(_G, _N, _N)
    row2 = lambda x: x.reshape(1, -1)

    adj, h = pl.pallas_call(
        _stage_a_kernel,
        grid=(_NBLK,),
        in_specs=[
            pl.BlockSpec((_GB, _N, _N), lambda i: (i, 0, 0)),
            pl.BlockSpec((_GB, _N, _N), lambda i: (i, 0, 0)),
            pl.BlockSpec((_N, _N), lambda i: (0, 0)),
            pl.BlockSpec((1, _N), lambda i: (0, 0)),
        ],
        out_specs=[
            pl.BlockSpec((_GB, _N, _N), lambda i: (i, 0, 0)),
            pl.BlockSpec((_GB * _N, _N), lambda i: (i, 0)),
        ],
        out_shape=[
            jax.ShapeDtypeStruct((_G, _N, _N), _f32),
            jax.ShapeDtypeStruct((_ROWS, _N), _f32),
        ],
        compiler_params=_SEQ,
    )(a3, v3, init_w, row2(init_b))

    blk_rows = pl.BlockSpec((_GB * _N, _N), lambda i: (i, 0))
    blk_adj = pl.BlockSpec((_GB, _N, _N), lambda i: (i, 0, 0))
    blk_w = pl.BlockSpec((_N, _N), lambda i: (0, 0))
    blk_r = pl.BlockSpec((1, _N), lambda i: (0, 0))
    blk_s = pl.BlockSpec((1, 1), lambda i: (0, 0))
    blk_g8 = pl.BlockSpec((_GB, _N), lambda i: (i, 0))

    logits = []
    latents = []
    ro_sum = None

    for l in range(_L):
        z1, s1, q1 = pl.pallas_call(
            _gin1_kernel,
            grid=(_NBLK,),
            in_specs=[blk_adj, blk_rows, blk_s, blk_w, blk_r],
            out_specs=[blk_rows, blk_r, blk_r],
            out_shape=[
                jax.ShapeDtypeStruct((_ROWS, _N), _f32),
                jax.ShapeDtypeStruct((1, _N), _f32),
                jax.ShapeDtypeStruct((1, _N), _f32),
            ],
            compiler_params=_SEQ,
        )(adj, h, gin_eps[l].reshape(1, 1), gin_w1[l], row2(gin_b1[l]))

        z2, s2, q2 = pl.pallas_call(
            _gin2_kernel,
            grid=(_NBLK,),
            in_specs=[blk_rows, blk_r, blk_r, blk_r, blk_r, blk_w, blk_r],
            out_specs=[blk_rows, blk_r, blk_r],
            out_shape=[
                jax.ShapeDtypeStruct((_ROWS, _N), _f32),
                jax.ShapeDtypeStruct((1, _N), _f32),
                jax.ShapeDtypeStruct((1, _N), _f32),
            ],
            compiler_params=_SEQ,
        )(z1, s1, q1, row2(gin_g1[l]), row2(gin_be1[l]),
          gin_w2[l], row2(gin_b2[l]))

        h, xr, hm, ro = pl.pallas_call(
            _gin3_kernel,
            grid=(_NBLK,),
            in_specs=[blk_rows, blk_r, blk_r, blk_r, blk_r],
            out_specs=[blk_rows, blk_g8, blk_g8, blk_s],
            out_shape=[
                jax.ShapeDtypeStruct((_ROWS, _N), _f32),
                jax.ShapeDtypeStruct((_G, _N), _f32),
                jax.ShapeDtypeStruct((_G, _N), _f32),
                jax.ShapeDtypeStruct((1, 1), _f32),
            ],
            compiler_params=_SEQ,
        )(z2, s2, q2, row2(gin_g2[l]), row2(gin_be2[l]))

        hro = pl.pallas_call(
            _sero_kernel,
            in_specs=[_full_spec((_G, _N)), _full_spec((_G, _N)),
                      _full_spec((_N, _N)), _full_spec((1, _N)),
                      _full_spec((1, _N)), _full_spec((1, _N)),
                      _full_spec((_N, _N)), _full_spec((1, _N))],
            out_specs=_full_spec((_G, _N)),
            out_shape=jax.ShapeDtypeStruct((_G, _N), _f32),
        )(xr, hm, sero_ew[l], row2(sero_eb[l]), row2(sero_g[l]),
          row2(sero_be[l]), sero_aw[l], row2(sero_ab[l]))

        inw = tr_inw[l]
        inb = tr_inb[l]
        ow = tr_ow[l]
        args = [hro]
        for base in (0, _N, 2 * _N):
            for hd in range(_HEADS):
                args.append(inw[base + hd * _HD: base + (hd + 1) * _HD, :])
        for base in (0, _N, 2 * _N):
            for hd in range(_HEADS):
                args.append(inb[base + hd * _HD: base + (hd + 1) * _HD]
                            .reshape(1, _HD))
        for hd in range(_HEADS):
            args.append(ow[:, hd * _HD:(hd + 1) * _HD])
        args += [row2(tr_ob[l]), row2(tr_ln1g[l]), row2(tr_ln1b[l]),
                 tr_m1w[l], row2(tr_m1b[l]), tr_m2w[l], row2(tr_m2b[l]),
                 row2(tr_ln2g[l]), row2(tr_ln2b[l]), cls_w[l],
                 row2(cls_b[l])]
        lat, lg = pl.pallas_call(
            _tr_kernel,
            in_specs=[_full_spec(tuple(x.shape)) for x in args],
            out_specs=[_full_spec((_B, _N)), _full_spec((_B, 2))],
            out_shape=[
                jax.ShapeDtypeStruct((_B, _N), _f32),
                jax.ShapeDtypeStruct((_B, 2), _f32),
            ],
        )(*args)

        logits.append(lg)
        latents.append(lat)
        ro_sum = ro if ro_sum is None else ro_sum + ro

    logit = logits[0] + logits[1] + logits[2] + logits[3]
    latent = jnp.stack(latents, axis=1)
    reg_ortho = (ro_sum * (1.0 / _G)).reshape(())
    return logit, latent, reg_ortho


# speedup vs baseline: 5.0731x; 1.5413x over previous
import numpy as np
import jax
import jax.numpy as jnp
from jax.experimental import pallas as pl
from jax.experimental.pallas import tpu as pltpu

_B, _T, _N, _H = 4, 64, 111, 111
_L = 4
_HEADS = 3
_HD = _N // _HEADS
_G = _B * _T
_GB = 8
_NBLK = _G // _GB
_RB = _GB * _N
_ROWS = _G * _N
_K = 8624
_EPS_BN = 1e-5
_P = 1 + 3 * _L

_f32 = jnp.float32


def _dotT(x, w):
    return jax.lax.dot_general(x, w, (((1,), (1,)), ((), ())),
                               preferred_element_type=_f32)


def _dot(x, w):
    return jax.lax.dot_general(x, w, (((1,), (0,)), ((), ())),
                               preferred_element_type=_f32)


def _mega_kernel(a_ref, v_ref, initw_ref, initb_ref, eps_ref,
                 w1_ref, b1_ref, g1_ref, be1_ref,
                 w2_ref, b2_ref, g2_ref, be2_ref,
                 ew_ref, eb_ref, sg_ref, sbe_ref, aw_ref, ab_ref,
                 wqkv_ref, bqkv_ref, owsl_ref, ob_ref,
                 ln1g_ref, ln1b_ref, m1w_ref, m1b_ref, m2w_ref, m2b_ref,
                 ln2g_ref, ln2b_ref, cw_ref, cb_ref,
                 lat_ref, log_ref, ro_ref,
                 adj_s, h_s, z_s, xr_s, hm_s,
                 s1_s, q1_s, s2_s, q2_s, ro_s, lg_s):
    p = pl.program_id(0)
    i = pl.program_id(1)
    ph = (p - 1) % 3
    is_g1 = (p >= 1) & (ph == 0)
    is_g2 = (p >= 1) & (ph == 1)
    is_g3 = (p >= 1) & (ph == 2)

    @pl.when(p == 0)
    def _stage_a():
        a = a_ref[...]
        i32 = jax.lax.bitcast_convert_type(a, jnp.int32)
        sign = jnp.int32(-2147483648)
        wb = jnp.where(i32 < 0, ~i32, i32 ^ sign)
        k = jnp.full((_GB, 1, 1), float(_K), _f32)
        pfx = jnp.zeros((_GB, 1, 1), jnp.int32)
        for bit in range(31, -1, -1):
            hi = jax.lax.shift_right_logical(wb, bit)
            tgt = jax.lax.shift_right_logical(pfx, bit)
            match0 = (hi == tgt).astype(_f32)
            c0 = jnp.sum(jnp.sum(match0, axis=2, keepdims=True),
                         axis=1, keepdims=True)
            go1 = k >= c0
            bitval = jnp.int32(np.int32(np.uint32(1 << bit)))
            pfx = jnp.where(go1, pfx | bitval, pfx)
            k = jnp.where(go1, k - c0, k)
        key = pfx ^ sign
        thr_i = jnp.where(key < 0, key ^ jnp.int32(0x7FFFFFFF), key)
        thr = jax.lax.bitcast_convert_type(thr_i, _f32)
        adj_s[i] = (a > thr).astype(_f32)
        w = initw_ref[...]
        b = initb_ref[...]
        for g in range(_GB):
            h_s[i, g * _N:(g + 1) * _N, :] = _dotT(v_ref[g], w) + b

    @pl.when(is_g1)
    def _g1():
        h = h_s[i]
        parts = []
        for g in range(_GB):
            parts.append(_dot(adj_s[i, g], h[g * _N:(g + 1) * _N, :]))
        agg = jnp.concatenate(parts, axis=0) + eps_ref[0, 0, 0] * h
        z = _dotT(agg, w1_ref[0]) + b1_ref[0]
        z_s[i] = z
        s = jnp.sum(z, axis=0, keepdims=True)
        q = jnp.sum(z * z, axis=0, keepdims=True)

        @pl.when(i == 0)
        def _init():
            s1_s[...] = s
            q1_s[...] = q

        @pl.when(i != 0)
        def _acc():
            s1_s[...] += s
            q1_s[...] += q

    @pl.when(is_g2)
    def _g2():
        m = s1_s[...] * (1.0 / _ROWS)
        var = q1_s[...] * (1.0 / _ROWS) - m * m
        y = (z_s[i] - m) / jnp.sqrt(var + _EPS_BN) * g1_ref[0] + be1_ref[0]
        y = jnp.maximum(y, 0.0)
        z2 = _dotT(y, w2_ref[0]) + b2_ref[0]
        z_s[i] = z2
        s = jnp.sum(z2, axis=0, keepdims=True)
        q = jnp.sum(z2 * z2, axis=0, keepdims=True)

        @pl.when(i == 0)
        def _init():
            s2_s[...] = s
            q2_s[...] = q

        @pl.when(i != 0)
        def _acc():
            s2_s[...] += s
            q2_s[...] += q

    @pl.when(is_g3)
    def _g3():
        m = s2_s[...] * (1.0 / _ROWS)
        var = q2_s[...] * (1.0 / _ROWS) - m * m
        h = (z_s[i] - m) / jnp.sqrt(var + _EPS_BN) * g2_ref[0] + be2_ref[0]
        h = jnp.maximum(h, 0.0)
        h_s[i] = h

        row = jax.lax.broadcasted_iota(jnp.int32, (_N, _N), 0)
        col = jax.lax.broadcasted_iota(jnp.int32, (_N, _N), 1)
        upper = col >= row
        eye = (col == row).astype(_f32)
        ones_row = jnp.ones((1, _N), _f32)

        xrs = []
        hms = []
        ro = jnp.zeros((1, 1), _f32)
        for g in range(_GB):
            hg = h[g * _N:(g + 1) * _N, :]
            xrs.append(_dot(ones_row, hg) * (1.0 / _N))
            hms.append(_dotT(ones_row, hg) * (1.0 / _N))
            mi = _dotT(hg, hg)
            mx = jnp.max(mi, axis=1, keepdims=True)
            mi_n = mi / mx
            diff = jnp.where(upper, mi_n - eye, 0.0)
            ssq = jnp.sum(jnp.sum(diff * diff, axis=1, keepdims=True),
                          axis=0, keepdims=True)
            ro = ro + jnp.sqrt(ssq)
        xr_s[pl.ds(i * _GB, _GB), :] = jnp.concatenate(xrs, axis=0)
        hm_s[pl.ds(i * _GB, _GB), :] = jnp.concatenate(hms, axis=0)

        first = (p == 3) & (i == 0)

        @pl.when(first)
        def _init():
            ro_s[...] = ro

        @pl.when(jnp.logical_not(first))
        def _acc():
            ro_s[...] += ro

        @pl.when(i == _NBLK - 1)
        def _tail():
            xr = xr_s[...]
            x = _dotT(xr, ew_ref[0]) + eb_ref[0]
            mm = jnp.mean(x, axis=0, keepdims=True)
            vv = jnp.mean((x - mm) * (x - mm), axis=0, keepdims=True)
            x = (x - mm) / jnp.sqrt(vv + _EPS_BN) * sg_ref[0] + sbe_ref[0]
            x = 0.5 * x * (1.0 + jax.lax.erf(x * np.float32(1.0 / np.sqrt(2.0))))
            gatt = jax.nn.sigmoid(_dotT(x, aw_ref[0]) + ab_ref[0])
            hro = gatt * hm_s[...]

            scale = np.float32(1.0 / np.sqrt(_HD))

            def _ln(x, g, b):
                mu = jnp.mean(x, axis=1, keepdims=True)
                va = jnp.mean((x - mu) * (x - mu), axis=1, keepdims=True)
                return (x - mu) / jnp.sqrt(va + _EPS_BN) * g + b

            lgs = []
            for b in range(_B):
                xb = hro[b * _T:(b + 1) * _T, :]
                att = jnp.zeros((_T, _N), _f32)
                for hd in range(_HEADS):
                    q = (_dotT(xb, wqkv_ref[hd, 0]) + bqkv_ref[hd, 0]) * scale
                    kk = _dotT(xb, wqkv_ref[3 + hd, 0]) + bqkv_ref[3 + hd, 0]
                    vvh = _dotT(xb, wqkv_ref[6 + hd, 0]) + bqkv_ref[6 + hd, 0]
                    sc = _dotT(q, kk)
                    mx = jnp.max(sc, axis=1, keepdims=True)
                    e = jnp.exp(sc - mx)
                    pa = e / jnp.sum(e, axis=1, keepdims=True)
                    o = _dot(pa, vvh)
                    att = att + jax.lax.dot_general(
                        o, owsl_ref[hd, 0], (((1,), (1,)), ((), ())),
                        preferred_element_type=_f32)
                att = att + ob_ref[0]
                x1 = _ln(att, ln1g_ref[0], ln1b_ref[0])
                x2 = jnp.maximum(_dotT(x1, m1w_ref[0]) + m1b_ref[0], 0.0)
                x2 = _dotT(x2, m2w_ref[0]) + m2b_ref[0]
                xo = _ln(x1 + x2, ln2g_ref[0], ln2b_ref[0])
                lat = jnp.sum(xo, axis=0, keepdims=True)
                lat_ref[0, b:b + 1, :] = lat
                lgs.append(_dotT(lat, cw_ref[0]) + cb_ref[0])
            lgc = jnp.concatenate(lgs, axis=0)

            @pl.when(p == 3)
            def _lg_init():
                lg_s[...] = lgc

            @pl.when(p != 3)
            def _lg_acc():
                lg_s[...] += lgc

            log_ref[...] = lg_s[...]
            ro_ref[...] = ro_s[...]


def kernel(v, a, init_w, init_b, gin_eps, gin_w1, gin_b1, gin_g1, gin_be1,
           gin_w2, gin_b2, gin_g2, gin_be2, sero_ew, sero_eb, sero_g, sero_be,
           sero_aw, sero_ab, tr_inw, tr_inb, tr_ow, tr_ob, tr_ln1g, tr_ln1b,
           tr_ln2g, tr_ln2b, tr_m1w, tr_m1b, tr_m2w, tr_m2b, cls_w, cls_b):
    a3 = a.reshape(_G, _N, _N)
    v3 = v.reshape(_G, _N, _N)
    r3 = lambda x: x.reshape(x.shape[0], 1, -1)

    wqkv = jnp.stack([tr_inw[:, base + hd * _HD: base + (hd + 1) * _HD, :]
                      for base in (0, _N, 2 * _N) for hd in range(_HEADS)])
    bqkv = jnp.stack([tr_inb[:, base + hd * _HD: base + (hd + 1) * _HD]
                      for base in (0, _N, 2 * _N)
                      for hd in range(_HEADS)])[:, :, None, :]
    owsl = jnp.stack([tr_ow[:, :, hd * _HD:(hd + 1) * _HD]
                      for hd in range(_HEADS)])

    def _lmap(p, i):
        return jnp.clip((p - 1) // 3, 0, _L - 1)

    def im_av(p, i):
        return (jnp.where(p == 0, i, 0), 0, 0)

    def im_const2(p, i):
        return (0, 0)

    def im_l3(p, i):
        return (_lmap(p, i), 0, 0)

    def im_l2(p, i):
        return (_lmap(p, i), 0)

    def im_l4(p, i):
        return (0, _lmap(p, i), 0, 0)

    in_specs = [
        pl.BlockSpec((_GB, _N, _N), im_av),
        pl.BlockSpec((_GB, _N, _N), im_av),
        pl.BlockSpec((_N, _N), im_const2),
        pl.BlockSpec((1, _N), im_const2),
        pl.BlockSpec((1, 1, 1), im_l3),
        pl.BlockSpec((1, _N, _N), im_l3),
        pl.BlockSpec((1, 1, _N), im_l3),
        pl.BlockSpec((1, 1, _N), im_l3),
        pl.BlockSpec((1, 1, _N), im_l3),
        pl.BlockSpec((1, _N, _N), im_l3),
        pl.BlockSpec((1, 1, _N), im_l3),
        pl.BlockSpec((1, 1, _N), im_l3),
        pl.BlockSpec((1, 1, _N), im_l3),
        pl.BlockSpec((1, _N, _N), im_l3),
        pl.BlockSpec((1, 1, _N), im_l3),
        pl.BlockSpec((1, 1, _N), im_l3),
        pl.BlockSpec((1, 1, _N), im_l3),
        pl.BlockSpec((1, _N, _N), im_l3),
        pl.BlockSpec((1, 1, _N), im_l3),
        pl.BlockSpec((9, 1, _HD, _N), im_l4),
        pl.BlockSpec((9, 1, 1, _HD), im_l4),
        pl.BlockSpec((3, 1, _N, _HD), im_l4),
        pl.BlockSpec((1, 1, _N), im_l3),
        pl.BlockSpec((1, 1, _N), im_l3),
        pl.BlockSpec((1, 1, _N), im_l3),
        pl.BlockSpec((1, 2 * _H, _N), im_l3),
        pl.BlockSpec((1, 1, 2 * _H), im_l3),
        pl.BlockSpec((1, _N, 2 * _H), im_l3),
        pl.BlockSpec((1, 1, _N), im_l3),
        pl.BlockSpec((1, 1, _N), im_l3),
        pl.BlockSpec((1, 1, _N), im_l3),
        pl.BlockSpec((1, 2, _N), im_l3),
        pl.BlockSpec((1, 1, 2), im_l3),
    ]
    out_specs = [
        pl.BlockSpec((1, _B, _N), lambda p, i: (_lmap(p, i), 0, 0)),
        pl.BlockSpec((_B, 2), im_const2),
        pl.BlockSpec((1, 1), im_const2),
    ]
    out_shape = [
        jax.ShapeDtypeStruct((_L, _B, _N), _f32),
        jax.ShapeDtypeStruct((_B, 2), _f32),
        jax.ShapeDtypeStruct((1, 1), _f32),
    ]
    scratch_shapes = [
        pltpu.VMEM((_NBLK, _GB, _N, _N), _f32),
        pltpu.VMEM((_NBLK, _RB, _N), _f32),
        pltpu.VMEM((_NBLK, _RB, _N), _f32),
        pltpu.VMEM((_G, _N), _f32),
        pltpu.VMEM((_G, _N), _f32),
        pltpu.VMEM((1, _N), _f32),
        pltpu.VMEM((1, _N), _f32),
        pltpu.VMEM((1, _N), _f32),
        pltpu.VMEM((1, _N), _f32),
        pltpu.VMEM((1, 1), _f32),
        pltpu.VMEM((_B, 2), _f32),
    ]

    lat, logit, ro = pl.pallas_call(
        _mega_kernel,
        grid=(_P, _NBLK),
        in_specs=in_specs,
        out_specs=out_specs,
        out_shape=out_shape,
        scratch_shapes=scratch_shapes,
        compiler_params=pltpu.CompilerParams(
            dimension_semantics=("arbitrary", "arbitrary")),
    )(a3, v3, init_w, init_b.reshape(1, _N), gin_eps.reshape(_L, 1, 1),
      gin_w1, r3(gin_b1), r3(gin_g1), r3(gin_be1),
      gin_w2, r3(gin_b2), r3(gin_g2), r3(gin_be2),
      sero_ew, r3(sero_eb), r3(sero_g), r3(sero_be), sero_aw, r3(sero_ab),
      wqkv, bqkv, owsl, r3(tr_ob), r3(tr_ln1g), r3(tr_ln1b),
      tr_m1w, r3(tr_m1b), tr_m2w, r3(tr_m2b), r3(tr_ln2g), r3(tr_ln2b),
      cls_w, r3(cls_b))

    return logit, jnp.transpose(lat, (1, 0, 2)), ro.reshape(()) * (1.0 / _G)


# speedup vs baseline: 7.4206x; 1.4627x over previous
import numpy as np
import jax
import jax.numpy as jnp
from jax.experimental import pallas as pl
from jax.experimental.pallas import tpu as pltpu

_B, _T, _N, _H = 4, 64, 111, 111
_L = 4
_HEADS = 3
_HD = _N // _HEADS
_G = _B * _T
_GB = 32
_NBLK = _G // _GB
_RB = _GB * _N
_ROWS = _G * _N
_K = 8624
_EPS_BN = 1e-5
_P = 1 + 3 * _L

_f32 = jnp.float32

_B1, _G1, _BE1, _B2, _G2, _BE2 = 0, 1, 2, 3, 4, 5
_EB, _SG, _SBE, _AB = 6, 7, 8, 9
_OB, _LN1G, _LN1B, _M2B, _LN2G, _LN2B = 10, 11, 12, 13, 14, 15


def _dotT(x, w):
    return jax.lax.dot_general(x, w, (((1,), (1,)), ((), ())),
                               preferred_element_type=_f32)


def _dot(x, w):
    return jax.lax.dot_general(x, w, (((1,), (0,)), ((), ())),
                               preferred_element_type=_f32)


def _mega_kernel(a_ref, v_ref, initw_ref, initb_ref, eps_ref,
                 wmat_ref, wrow_ref, m1w_ref, m1b_ref, m2w_ref,
                 wqkv_ref, bqkv_ref, owsl_ref, cw_ref, cb_ref,
                 lat_ref, log_ref, ro_ref,
                 adj_s, h_s, z_s, xr_s, hm_s,
                 s1_s, q1_s, s2_s, q2_s, ro_s, lg_s):
    p = pl.program_id(0)
    i = pl.program_id(1)
    ph = (p - 1) % 3
    is_g1 = (p >= 1) & (ph == 0)
    is_g2 = (p >= 1) & (ph == 1)
    is_g3 = (p >= 1) & (ph == 2)

    def row(j):
        return wrow_ref[0, j:j + 1, :]

    @pl.when(p == 0)
    def _stage_a():
        a = a_ref[...]
        i32 = jax.lax.bitcast_convert_type(a, jnp.int32)
        sign = jnp.int32(-2147483648)
        wb = jnp.where(i32 < 0, ~i32, i32 ^ sign)
        k = jnp.full((_GB, 1, 1), float(_K), _f32)
        pfx = jnp.zeros((_GB, 1, 1), jnp.int32)
        for bit in range(31, -1, -1):
            hi = jax.lax.shift_right_logical(wb, bit)
            tgt = jax.lax.shift_right_logical(pfx, bit)
            match0 = (hi == tgt).astype(_f32)
            c0 = jnp.sum(jnp.sum(match0, axis=2, keepdims=True),
                         axis=1, keepdims=True)
            go1 = k >= c0
            bitval = jnp.int32(np.int32(np.uint32(1 << bit)))
            pfx = jnp.where(go1, pfx | bitval, pfx)
            k = jnp.where(go1, k - c0, k)
        key = pfx ^ sign
        thr_i = jnp.where(key < 0, key ^ jnp.int32(0x7FFFFFFF), key)
        thr = jax.lax.bitcast_convert_type(thr_i, _f32)
        adj_s[i] = (a > thr).astype(_f32)
        w = initw_ref[...]
        b = initb_ref[...]
        for g in range(_GB):
            h_s[i, g * _N:(g + 1) * _N, :] = _dotT(v_ref[g], w) + b

    @pl.when(is_g1)
    def _g1():
        h = h_s[i]
        parts = []
        for g in range(_GB):
            parts.append(_dot(adj_s[i, g], h[g * _N:(g + 1) * _N, :]))
        agg = jnp.concatenate(parts, axis=0) + eps_ref[0, 0, 0] * h
        z = _dotT(agg, wmat_ref[0, 0]) + row(_B1)
        z_s[i] = z
        s = jnp.sum(z, axis=0, keepdims=True)
        q = jnp.sum(z * z, axis=0, keepdims=True)

        @pl.when(i == 0)
        def _init():
            s1_s[...] = s
            q1_s[...] = q

        @pl.when(i != 0)
        def _acc():
            s1_s[...] += s
            q1_s[...] += q

    @pl.when(is_g2)
    def _g2():
        m = s1_s[...] * (1.0 / _ROWS)
        var = q1_s[...] * (1.0 / _ROWS) - m * m
        y = (z_s[i] - m) / jnp.sqrt(var + _EPS_BN) * row(_G1) + row(_BE1)
        y = jnp.maximum(y, 0.0)
        z2 = _dotT(y, wmat_ref[0, 1]) + row(_B2)
        z_s[i] = z2
        s = jnp.sum(z2, axis=0, keepdims=True)
        q = jnp.sum(z2 * z2, axis=0, keepdims=True)

        @pl.when(i == 0)
        def _init():
            s2_s[...] = s
            q2_s[...] = q

        @pl.when(i != 0)
        def _acc():
            s2_s[...] += s
            q2_s[...] += q

    @pl.when(is_g3)
    def _g3():
        m = s2_s[...] * (1.0 / _ROWS)
        var = q2_s[...] * (1.0 / _ROWS) - m * m
        h = (z_s[i] - m) / jnp.sqrt(var + _EPS_BN) * row(_G2) + row(_BE2)
        h = jnp.maximum(h, 0.0)
        h_s[i] = h

        rr = jax.lax.broadcasted_iota(jnp.int32, (_N, _N), 0)
        cc = jax.lax.broadcasted_iota(jnp.int32, (_N, _N), 1)
        upper = cc >= rr
        eye = (cc == rr).astype(_f32)
        ones_row = jnp.ones((1, _N), _f32)

        xrs = []
        hms = []
        ro = jnp.zeros((1, 1), _f32)
        for g in range(_GB):
            hg = h[g * _N:(g + 1) * _N, :]
            xrs.append(_dot(ones_row, hg) * (1.0 / _N))
            hms.append(_dotT(ones_row, hg) * (1.0 / _N))
            mi = _dotT(hg, hg)
            mx = jnp.max(mi, axis=1, keepdims=True)
            mi_n = mi / mx
            diff = jnp.where(upper, mi_n - eye, 0.0)
            ssq = jnp.sum(jnp.sum(diff * diff, axis=1, keepdims=True),
                          axis=0, keepdims=True)
            ro = ro + jnp.sqrt(ssq)
        xr_s[pl.ds(i * _GB, _GB), :] = jnp.concatenate(xrs, axis=0)
        hm_s[pl.ds(i * _GB, _GB), :] = jnp.concatenate(hms, axis=0)

        first = (p == 3) & (i == 0)

        @pl.when(first)
        def _init():
            ro_s[...] = ro

        @pl.when(jnp.logical_not(first))
        def _acc():
            ro_s[...] += ro

        @pl.when(i == _NBLK - 1)
        def _tail():
            xr = xr_s[...]
            x = _dotT(xr, wmat_ref[0, 2]) + row(_EB)
            mm = jnp.mean(x, axis=0, keepdims=True)
            vv = jnp.mean((x - mm) * (x - mm), axis=0, keepdims=True)
            x = (x - mm) / jnp.sqrt(vv + _EPS_BN) * row(_SG) + row(_SBE)
            x = 0.5 * x * (1.0 + jax.lax.erf(x * np.float32(1.0 / np.sqrt(2.0))))
            gatt = jax.nn.sigmoid(_dotT(x, wmat_ref[0, 3]) + row(_AB))
            hro = gatt * hm_s[...]

            scale = np.float32(1.0 / np.sqrt(_HD))

            def _ln(x, g, b):
                mu = jnp.mean(x, axis=1, keepdims=True)
                va = jnp.mean((x - mu) * (x - mu), axis=1, keepdims=True)
                return (x - mu) / jnp.sqrt(va + _EPS_BN) * g + b

            lgs = []
            for b in range(_B):
                xb = hro[b * _T:(b + 1) * _T, :]
                att = jnp.zeros((_T, _N), _f32)
                for hd in range(_HEADS):
                    q = (_dotT(xb, wqkv_ref[hd, 0]) + bqkv_ref[hd, 0]) * scale
                    kk = _dotT(xb, wqkv_ref[3 + hd, 0]) + bqkv_ref[3 + hd, 0]
                    vvh = _dotT(xb, wqkv_ref[6 + hd, 0]) + bqkv_ref[6 + hd, 0]
                    sc = _dotT(q, kk)
                    mx = jnp.max(sc, axis=1, keepdims=True)
                    e = jnp.exp(sc - mx)
                    pa = e / jnp.sum(e, axis=1, keepdims=True)
                    o = _dot(pa, vvh)
                    att = att + jax.lax.dot_general(
                        o, owsl_ref[hd, 0], (((1,), (1,)), ((), ())),
                        preferred_element_type=_f32)
                att = att + row(_OB)
                x1 = _ln(att, row(_LN1G), row(_LN1B))
                x2 = jnp.maximum(_dotT(x1, m1w_ref[0]) + m1b_ref[0], 0.0)
                x2 = _dotT(x2, m2w_ref[0]) + row(_M2B)
                xo = _ln(x1 + x2, row(_LN2G), row(_LN2B))
                lat = jnp.sum(xo, axis=0, keepdims=True)
                lat_ref[0, b:b + 1, :] = lat
                lgs.append(_dotT(lat, cw_ref[0]) + cb_ref[0])
            lgc = jnp.concatenate(lgs, axis=0)

            @pl.when(p == 3)
            def _lg_init():
                lg_s[...] = lgc

            @pl.when(p != 3)
            def _lg_acc():
                lg_s[...] += lgc

            log_ref[...] = lg_s[...]
            ro_ref[...] = ro_s[...]


def kernel(v, a, init_w, init_b, gin_eps, gin_w1, gin_b1, gin_g1, gin_be1,
           gin_w2, gin_b2, gin_g2, gin_be2, sero_ew, sero_eb, sero_g, sero_be,
           sero_aw, sero_ab, tr_inw, tr_inb, tr_ow, tr_ob, tr_ln1g, tr_ln1b,
           tr_ln2g, tr_ln2b, tr_m1w, tr_m1b, tr_m2w, tr_m2b, cls_w, cls_b):
    a3 = a.reshape(_G, _N, _N)
    v3 = v.reshape(_G, _N, _N)

    wmat = jnp.stack([gin_w1, gin_w2, sero_ew, sero_aw], axis=1)
    wrow = jnp.stack([gin_b1, gin_g1, gin_be1, gin_b2, gin_g2, gin_be2,
                      sero_eb, sero_g, sero_be, sero_ab,
                      tr_ob, tr_ln1g, tr_ln1b, tr_m2b, tr_ln2g, tr_ln2b],
                     axis=1)
    wqkv = jnp.stack([tr_inw[:, base + hd * _HD: base + (hd + 1) * _HD, :]
                      for base in (0, _N, 2 * _N) for hd in range(_HEADS)])
    bqkv = jnp.stack([tr_inb[:, base + hd * _HD: base + (hd + 1) * _HD]
                      for base in (0, _N, 2 * _N)
                      for hd in range(_HEADS)])[:, :, None, :]
    owsl = jnp.stack([tr_ow[:, :, hd * _HD:(hd + 1) * _HD]
                      for hd in range(_HEADS)])

    def _lmap(p, i):
        return jnp.clip((p - 1) // 3, 0, _L - 1)

    def im_av(p, i):
        return (jnp.where(p == 0, i, 0), 0, 0)

    def im_const2(p, i):
        return (0, 0)

    def im_l3(p, i):
        return (_lmap(p, i), 0, 0)

    def im_l4(p, i):
        return (0, _lmap(p, i), 0, 0)

    def im_l4a(p, i):
        return (_lmap(p, i), 0, 0, 0)

    in_specs = [
        pl.BlockSpec((_GB, _N, _N), im_av),
        pl.BlockSpec((_GB, _N, _N), im_av),
        pl.BlockSpec((_N, _N), im_const2),
        pl.BlockSpec((1, _N), im_const2),
        pl.BlockSpec((1, 1, 1), im_l3),
        pl.BlockSpec((1, 4, _N, _N), im_l4a),
        pl.BlockSpec((1, 16, _N), im_l3),
        pl.BlockSpec((1, 2 * _H, _N), im_l3),
        pl.BlockSpec((1, 1, 2 * _H), im_l3),
        pl.BlockSpec((1, _N, 2 * _H), im_l3),
        pl.BlockSpec((9, 1, _HD, _N), im_l4),
        pl.BlockSpec((9, 1, 1, _HD), im_l4),
        pl.BlockSpec((3, 1, _N, _HD), im_l4),
        pl.BlockSpec((1, 2, _N), im_l3),
        pl.BlockSpec((1, 1, 2), im_l3),
    ]
    out_specs = [
        pl.BlockSpec((1, _B, _N), lambda p, i: (_lmap(p, i), 0, 0)),
        pl.BlockSpec((_B, 2), im_const2),
        pl.BlockSpec((1, 1), im_const2),
    ]
    out_shape = [
        jax.ShapeDtypeStruct((_L, _B, _N), _f32),
        jax.ShapeDtypeStruct((_B, 2), _f32),
        jax.ShapeDtypeStruct((1, 1), _f32),
    ]
    scratch_shapes = [
        pltpu.VMEM((_NBLK, _GB, _N, _N), _f32),
        pltpu.VMEM((_NBLK, _RB, _N), _f32),
        pltpu.VMEM((_NBLK, _RB, _N), _f32),
        pltpu.VMEM((_G, _N), _f32),
        pltpu.VMEM((_G, _N), _f32),
        pltpu.VMEM((1, _N), _f32),
        pltpu.VMEM((1, _N), _f32),
        pltpu.VMEM((1, _N), _f32),
        pltpu.VMEM((1, _N), _f32),
        pltpu.VMEM((1, 1), _f32),
        pltpu.VMEM((_B, 2), _f32),
    ]

    lat, logit, ro = pl.pallas_call(
        _mega_kernel,
        grid=(_P, _NBLK),
        in_specs=in_specs,
        out_specs=out_specs,
        out_shape=out_shape,
        scratch_shapes=scratch_shapes,
        compiler_params=pltpu.CompilerParams(
            dimension_semantics=("arbitrary", "arbitrary")),
    )(a3, v3, init_w, init_b.reshape(1, _N), gin_eps.reshape(_L, 1, 1),
      wmat, wrow, tr_m1w, tr_m1b[:, None, :], tr_m2w,
      wqkv, bqkv, owsl, cls_w, cls_b[:, None, :])

    return logit, jnp.transpose(lat, (1, 0, 2)), ro.reshape(()) * (1.0 / _G)
